# baseline (speedup 1.0000x reference)
import functools

import jax
import jax.numpy as jnp
from jax import lax
from jax.experimental import pallas as pl
from jax.experimental.pallas import tpu as pltpu
from jax.experimental.pallas import tpu_sc as plsc

ROWS = 64
VOCAB = 1_000_000
NUM_CORES = 2
NUM_SUBCORES = 16
NW = NUM_CORES * NUM_SUBCORES
SUBROWS = 8
GROUPS = 4
SPLIT = 475_136
TCOLS_PER_G = SPLIT // 128 // GROUPS
CHUNK_T = 48
CHUNK_W = CHUNK_T * 128
VECS = CHUNK_W // 16
CHUNK_STARTS = list(range(0, TCOLS_PER_G - CHUNK_T + 1, CHUNK_T))
if CHUNK_STARTS[-1] != TCOLS_PER_G - CHUNK_T:
    CHUNK_STARTS.append(TCOLS_PER_G - CHUNK_T)
BLK = 16384
OFF_BLK = SPLIT // BLK
TAIL_COL = 999_424
TC_STEPS = (TAIL_COL - SPLIT) // BLK
TAIL_W = VOCAB - TAIL_COL
INT_MAX = 2**31 - 1


def _lane_permute(x, perm):
    dnums = lax.GatherDimensionNumbers(
        offset_dims=(), collapsed_slice_dims=(0,), start_index_map=(0,))
    return lax.gather(x, perm[:, None], dnums, slice_sizes=(1,),
                      mode=lax.GatherScatterMode.PROMISE_IN_BOUNDS)


def _combine(v1, i1, v2, i2):
    p = (v2 > v1) | ((v2 == v1) & (i2 < i1))
    return jnp.where(p, v2, v1), jnp.where(p, i2, i1)


def _sc_body(logits_hbm, out_i_hbm, out_v_hbm, buf0, buf1, pbuf, acc_m,
             acc_i, res_i, res_v, tmp_v, tmp_i, spm_v, spm_i, sem0, sem1):
    cid = lax.axis_index("c")
    sid = lax.axis_index("s")
    wid = cid * NUM_SUBCORES + sid
    tile_row = wid // GROUPS
    g = wid % GROUPS
    row0 = tile_row * SUBROWS
    col_g = g * (TCOLS_PER_G * 128)
    bufs = (buf0, buf1)
    sems = (sem0, sem1)
    iota = lax.iota(jnp.int32, 16)

    neg_inf = jnp.full((16,), -jnp.inf, jnp.float32)
    for s in range(SUBROWS):
        acc_m[s, pl.ds(0, 16)] = neg_inf
        acc_i[s, pl.ds(0, 16)] = jnp.zeros((16,), jnp.int32)

    def start_dma(c):
        b = c % 2
        return pltpu.make_async_copy(
            logits_hbm.at[pl.ds(row0, SUBROWS),
                          pl.ds(col_g + CHUNK_STARTS[c] * 128, CHUNK_W)],
            bufs[b], sems[b])

    copies = [None, None]
    copies[0] = start_dma(0)
    copies[0].start()
    for c in range(len(CHUNK_STARTS)):
        if c + 1 < len(CHUNK_STARTS):
            copies[(c + 1) % 2] = start_dma(c + 1)
            copies[(c + 1) % 2].start()
        copies[c % 2].wait()
        buf = bufs[c % 2]
        col0 = col_g + CHUNK_STARTS[c] * 128

        def s_body(s, _, buf=buf, col0=col0):
            def body(j, carry):
                mv, mi, bi = carry
                v = buf[s, pl.ds(j * 16, 16)]
                p = v > mv
                return (jnp.where(p, v, mv), jnp.where(p, bi, mi), bi + 16)
            mv, mi, _ = lax.fori_loop(
                0, VECS, body,
                (acc_m[s, pl.ds(0, 16)], acc_i[s, pl.ds(0, 16)],
                 col0 + iota),
                unroll=8)
            acc_m[s, pl.ds(0, 16)] = mv
            acc_i[s, pl.ds(0, 16)] = mi
            return 0

        lax.fori_loop(0, SUBROWS, s_body, 0)

    @pl.when(g == GROUPS - 1)
    def _():
        pltpu.sync_copy(
            logits_hbm.at[pl.ds(row0, SUBROWS), pl.ds(TAIL_COL, TAIL_W)],
            pbuf)

        def ps_body(s, _):
            def body(j, carry):
                mv, mi, bi = carry
                v = pbuf[s, pl.ds(j * 16, 16)]
                p = v > mv
                return (jnp.where(p, v, mv), jnp.where(p, bi, mi), bi + 16)
            mv, mi, _ = lax.fori_loop(
                0, TAIL_W // 16, body,
                (acc_m[s, pl.ds(0, 16)], acc_i[s, pl.ds(0, 16)],
                 TAIL_COL + iota))
            acc_m[s, pl.ds(0, 16)] = mv
            acc_i[s, pl.ds(0, 16)] = mi
            return 0

        lax.fori_loop(0, SUBROWS, ps_body, 0)

    for s in range(SUBROWS):
        mv, mi = acc_m[s, pl.ds(0, 16)], acc_i[s, pl.ds(0, 16)]
        for sh in (8, 4, 2, 1):
            perm = (iota + sh) & 15
            mv, mi = _combine(mv, mi, _lane_permute(mv, perm),
                              _lane_permute(mi, perm))
        acc_m[s, pl.ds(0, 16)] = mv
        acc_i[s, pl.ds(0, 16)] = mi

    pltpu.sync_copy(acc_m, spm_v.at[sid])
    pltpu.sync_copy(acc_i, spm_i.at[sid])
    plsc.subcore_barrier()

    for t in range(2):
        lrow = sid * 2 + t
        rloc = lax.div(lrow, SUBROWS)
        s = lax.rem(lrow, SUBROWS)
        mv, mi = None, None
        for q in range(GROUPS):
            pltpu.sync_copy(spm_v.at[rloc * GROUPS + q], tmp_v)
            pltpu.sync_copy(spm_i.at[rloc * GROUPS + q], tmp_i)
            v = tmp_v[s, pl.ds(0, 16)]
            i = tmp_i[s, pl.ds(0, 16)]
            if mv is None:
                mv, mi = v, i
            else:
                mv, mi = _combine(mv, mi, v, i)
        res_i[...] = mi
        res_v[...] = mv
        off = (cid * 32 + lrow) * 16
        pltpu.sync_copy(res_i, out_i_hbm.at[pl.ds(off, 16)])
        pltpu.sync_copy(res_v, out_v_hbm.at[pl.ds(off, 16)])


def _tc_body(in_ref, val_ref, idx_ref):
    pid = pl.program_id(0)

    @pl.when(pid == 0)
    def _():
        val_ref[...] = jnp.full((ROWS,), -jnp.inf, jnp.float32)
        idx_ref[...] = jnp.zeros((ROWS,), jnp.int32)

    x = in_ref[...]
    bm = jnp.max(x, axis=1)
    bi = (pid + OFF_BLK) * BLK + jnp.argmax(x, axis=1).astype(jnp.int32)
    cur_v = val_ref[...]
    cur_i = idx_ref[...]
    p = bm > cur_v
    val_ref[...] = jnp.where(p, bm, cur_v)
    idx_ref[...] = jnp.where(p, bi, cur_i)


@jax.jit
def _argmax_split(logits):
    mesh = plsc.VectorSubcoreMesh(core_axis_name="c", subcore_axis_name="s")
    sc_run = pl.kernel(
        _sc_body,
        mesh=mesh,
        out_type=(jax.ShapeDtypeStruct((ROWS * 16,), jnp.int32),
                  jax.ShapeDtypeStruct((ROWS * 16,), jnp.float32)),
        scratch_types=[
            pltpu.VMEM((SUBROWS, CHUNK_W), jnp.float32),
            pltpu.VMEM((SUBROWS, CHUNK_W), jnp.float32),
            pltpu.VMEM((SUBROWS, TAIL_W), jnp.float32),
            pltpu.VMEM((SUBROWS, 128), jnp.float32),
            pltpu.VMEM((SUBROWS, 128), jnp.int32),
            pltpu.VMEM((16,), jnp.int32),
            pltpu.VMEM((16,), jnp.float32),
            pltpu.VMEM((SUBROWS, 128), jnp.float32),
            pltpu.VMEM((SUBROWS, 128), jnp.int32),
            pltpu.VMEM_SHARED((NUM_SUBCORES, SUBROWS, 128), jnp.float32),
            pltpu.VMEM_SHARED((NUM_SUBCORES, SUBROWS, 128), jnp.int32),
            pltpu.SemaphoreType.DMA,
            pltpu.SemaphoreType.DMA,
        ],
    )
    sc_i, sc_v = sc_run(logits)
    sc_i = sc_i[::16]
    sc_v = sc_v[::16]

    tc_v, tc_i = pl.pallas_call(
        _tc_body,
        grid=(TC_STEPS,),
        in_specs=[pl.BlockSpec((ROWS, BLK), lambda i: (0, i + OFF_BLK))],
        out_specs=(pl.BlockSpec((ROWS,), lambda i: (0,)),
                   pl.BlockSpec((ROWS,), lambda i: (0,))),
        out_shape=(jax.ShapeDtypeStruct((ROWS,), jnp.float32),
                   jax.ShapeDtypeStruct((ROWS,), jnp.int32)),
    )(logits)

    p = (tc_v > sc_v) | ((tc_v == sc_v) & (tc_i < sc_i))
    return jnp.where(p, tc_i, sc_i)


def kernel(logits):
    return _argmax_split(logits)


# speedup vs baseline: 1.0336x; 1.0336x over previous
import functools

import jax
import jax.numpy as jnp
from jax import lax
from jax.experimental import pallas as pl
from jax.experimental.pallas import tpu as pltpu
from jax.experimental.pallas import tpu_sc as plsc

ROWS = 64
VOCAB = 1_000_000
NUM_CORES = 2
NUM_SUBCORES = 16
NW = NUM_CORES * NUM_SUBCORES
SUBROWS = 8
GROUPS = 4
SPLIT = 458_752
TCOLS_PER_G = SPLIT // 128 // GROUPS
CHUNK_T = 48
CHUNK_W = CHUNK_T * 128
VECS = CHUNK_W // 16
CHUNK_STARTS = list(range(0, TCOLS_PER_G - CHUNK_T + 1, CHUNK_T))
if CHUNK_STARTS[-1] != TCOLS_PER_G - CHUNK_T:
    CHUNK_STARTS.append(TCOLS_PER_G - CHUNK_T)
BLK = 16384
OFF_BLK = SPLIT // BLK
TAIL_COL = 999_424
TC_STEPS = (TAIL_COL - SPLIT) // BLK
TAIL_W = VOCAB - TAIL_COL
INT_MAX = 2**31 - 1


def _lane_permute(x, perm):
    dnums = lax.GatherDimensionNumbers(
        offset_dims=(), collapsed_slice_dims=(0,), start_index_map=(0,))
    return lax.gather(x, perm[:, None], dnums, slice_sizes=(1,),
                      mode=lax.GatherScatterMode.PROMISE_IN_BOUNDS)


def _combine(v1, i1, v2, i2):
    p = (v2 > v1) | ((v2 == v1) & (i2 < i1))
    return jnp.where(p, v2, v1), jnp.where(p, i2, i1)


def _sc_body(logits_hbm, out_i_hbm, out_v_hbm, buf0, buf1, pbuf, acc_m,
             acc_i, res_i, res_v, tmp_v, tmp_i, spm_v, spm_i, sem0, sem1):
    cid = lax.axis_index("c")
    sid = lax.axis_index("s")
    wid = cid * NUM_SUBCORES + sid
    tile_row = wid // GROUPS
    g = wid % GROUPS
    row0 = tile_row * SUBROWS
    col_g = g * (TCOLS_PER_G * 128)
    bufs = (buf0, buf1)
    sems = (sem0, sem1)
    iota = lax.iota(jnp.int32, 16)

    neg_inf = jnp.full((16,), -jnp.inf, jnp.float32)
    for s in range(SUBROWS):
        acc_m[s, pl.ds(0, 16)] = neg_inf
        acc_i[s, pl.ds(0, 16)] = jnp.zeros((16,), jnp.int32)

    def start_dma(c):
        b = c % 2
        return pltpu.make_async_copy(
            logits_hbm.at[pl.ds(row0, SUBROWS),
                          pl.ds(col_g + CHUNK_STARTS[c] * 128, CHUNK_W)],
            bufs[b], sems[b])

    copies = [None, None]
    copies[0] = start_dma(0)
    copies[0].start()
    for c in range(len(CHUNK_STARTS)):
        if c + 1 < len(CHUNK_STARTS):
            copies[(c + 1) % 2] = start_dma(c + 1)
            copies[(c + 1) % 2].start()
        copies[c % 2].wait()
        buf = bufs[c % 2]
        col0 = col_g + CHUNK_STARTS[c] * 128

        def s_body(s, _, buf=buf, col0=col0):
            def body(j, carry):
                mv, mi, bi = carry
                v = buf[s, pl.ds(j * 16, 16)]
                p = v > mv
                return (jnp.where(p, v, mv), jnp.where(p, bi, mi), bi + 16)
            mv, mi, _ = lax.fori_loop(
                0, VECS, body,
                (acc_m[s, pl.ds(0, 16)], acc_i[s, pl.ds(0, 16)],
                 col0 + iota),
                unroll=8)
            acc_m[s, pl.ds(0, 16)] = mv
            acc_i[s, pl.ds(0, 16)] = mi
            return 0

        lax.fori_loop(0, SUBROWS, s_body, 0)

    @pl.when(g == GROUPS - 1)
    def _():
        pltpu.sync_copy(
            logits_hbm.at[pl.ds(row0, SUBROWS), pl.ds(TAIL_COL, TAIL_W)],
            pbuf)

        def ps_body(s, _):
            def body(j, carry):
                mv, mi, bi = carry
                v = pbuf[s, pl.ds(j * 16, 16)]
                p = v > mv
                return (jnp.where(p, v, mv), jnp.where(p, bi, mi), bi + 16)
            mv, mi, _ = lax.fori_loop(
                0, TAIL_W // 16, body,
                (acc_m[s, pl.ds(0, 16)], acc_i[s, pl.ds(0, 16)],
                 TAIL_COL + iota))
            acc_m[s, pl.ds(0, 16)] = mv
            acc_i[s, pl.ds(0, 16)] = mi
            return 0

        lax.fori_loop(0, SUBROWS, ps_body, 0)

    for s in range(SUBROWS):
        mv, mi = acc_m[s, pl.ds(0, 16)], acc_i[s, pl.ds(0, 16)]
        for sh in (8, 4, 2, 1):
            perm = (iota + sh) & 15
            mv, mi = _combine(mv, mi, _lane_permute(mv, perm),
                              _lane_permute(mi, perm))
        acc_m[s, pl.ds(0, 16)] = mv
        acc_i[s, pl.ds(0, 16)] = mi

    pltpu.sync_copy(acc_m, spm_v.at[sid])
    pltpu.sync_copy(acc_i, spm_i.at[sid])
    plsc.subcore_barrier()

    for t in range(2):
        lrow = sid * 2 + t
        rloc = lax.div(lrow, SUBROWS)
        s = lax.rem(lrow, SUBROWS)
        mv, mi = None, None
        for q in range(GROUPS):
            pltpu.sync_copy(spm_v.at[rloc * GROUPS + q], tmp_v)
            pltpu.sync_copy(spm_i.at[rloc * GROUPS + q], tmp_i)
            v = tmp_v[s, pl.ds(0, 16)]
            i = tmp_i[s, pl.ds(0, 16)]
            if mv is None:
                mv, mi = v, i
            else:
                mv, mi = _combine(mv, mi, v, i)
        res_i[...] = mi
        res_v[...] = mv
        off = (cid * 32 + lrow) * 16
        pltpu.sync_copy(res_i, out_i_hbm.at[pl.ds(off, 16)])
        pltpu.sync_copy(res_v, out_v_hbm.at[pl.ds(off, 16)])


def _tc_body(in_ref, val_ref, idx_ref):
    pid = pl.program_id(0)

    @pl.when(pid == 0)
    def _():
        val_ref[...] = jnp.full((ROWS,), -jnp.inf, jnp.float32)
        idx_ref[...] = jnp.zeros((ROWS,), jnp.int32)

    x = in_ref[...]
    bm = jnp.max(x, axis=1)
    bi = (pid + OFF_BLK) * BLK + jnp.argmax(x, axis=1).astype(jnp.int32)
    cur_v = val_ref[...]
    cur_i = idx_ref[...]
    p = bm > cur_v
    val_ref[...] = jnp.where(p, bm, cur_v)
    idx_ref[...] = jnp.where(p, bi, cur_i)


@jax.jit
def _argmax_split(logits):
    mesh = plsc.VectorSubcoreMesh(core_axis_name="c", subcore_axis_name="s")
    sc_run = pl.kernel(
        _sc_body,
        mesh=mesh,
        out_type=(jax.ShapeDtypeStruct((ROWS * 16,), jnp.int32),
                  jax.ShapeDtypeStruct((ROWS * 16,), jnp.float32)),
        scratch_types=[
            pltpu.VMEM((SUBROWS, CHUNK_W), jnp.float32),
            pltpu.VMEM((SUBROWS, CHUNK_W), jnp.float32),
            pltpu.VMEM((SUBROWS, TAIL_W), jnp.float32),
            pltpu.VMEM((SUBROWS, 128), jnp.float32),
            pltpu.VMEM((SUBROWS, 128), jnp.int32),
            pltpu.VMEM((16,), jnp.int32),
            pltpu.VMEM((16,), jnp.float32),
            pltpu.VMEM((SUBROWS, 128), jnp.float32),
            pltpu.VMEM((SUBROWS, 128), jnp.int32),
            pltpu.VMEM_SHARED((NUM_SUBCORES, SUBROWS, 128), jnp.float32),
            pltpu.VMEM_SHARED((NUM_SUBCORES, SUBROWS, 128), jnp.int32),
            pltpu.SemaphoreType.DMA,
            pltpu.SemaphoreType.DMA,
        ],
    )
    sc_i, sc_v = sc_run(logits)
    sc_i = sc_i[::16]
    sc_v = sc_v[::16]

    tc_v, tc_i = pl.pallas_call(
        _tc_body,
        grid=(TC_STEPS,),
        in_specs=[pl.BlockSpec((ROWS, BLK), lambda i: (0, i + OFF_BLK))],
        out_specs=(pl.BlockSpec((ROWS,), lambda i: (0,)),
                   pl.BlockSpec((ROWS,), lambda i: (0,))),
        out_shape=(jax.ShapeDtypeStruct((ROWS,), jnp.float32),
                   jax.ShapeDtypeStruct((ROWS,), jnp.int32)),
    )(logits)

    p = (tc_v > sc_v) | ((tc_v == sc_v) & (tc_i < sc_i))
    return jnp.where(p, tc_i, sc_i)


def kernel(logits):
    return _argmax_split(logits)


# speedup vs baseline: 1.0530x; 1.0187x over previous
import functools

import jax
import jax.numpy as jnp
from jax import lax
from jax.experimental import pallas as pl
from jax.experimental.pallas import tpu as pltpu
from jax.experimental.pallas import tpu_sc as plsc

ROWS = 64
VOCAB = 1_000_000
NUM_CORES = 2
NUM_SUBCORES = 16
NW = NUM_CORES * NUM_SUBCORES
SUBROWS = 8
GROUPS = 4
SPLIT = 466_944
TCOLS_PER_G = SPLIT // 128 // GROUPS
CHUNK_T = 48
CHUNK_W = CHUNK_T * 128
VECS = CHUNK_W // 16
CHUNK_STARTS = list(range(0, TCOLS_PER_G - CHUNK_T + 1, CHUNK_T))
if CHUNK_STARTS[-1] != TCOLS_PER_G - CHUNK_T:
    CHUNK_STARTS.append(TCOLS_PER_G - CHUNK_T)
BLK = 8192
OFF_BLK = SPLIT // BLK
TAIL_COL = 999_424
TC_STEPS = (TAIL_COL - SPLIT) // BLK
TAIL_W = VOCAB - TAIL_COL
INT_MAX = 2**31 - 1


def _lane_permute(x, perm):
    dnums = lax.GatherDimensionNumbers(
        offset_dims=(), collapsed_slice_dims=(0,), start_index_map=(0,))
    return lax.gather(x, perm[:, None], dnums, slice_sizes=(1,),
                      mode=lax.GatherScatterMode.PROMISE_IN_BOUNDS)


def _combine(v1, i1, v2, i2):
    p = (v2 > v1) | ((v2 == v1) & (i2 < i1))
    return jnp.where(p, v2, v1), jnp.where(p, i2, i1)


def _sc_body(logits_hbm, out_i_hbm, out_v_hbm, buf0, buf1, pbuf, acc_m,
             acc_i, res_i, res_v, tmp_v, tmp_i, spm_v, spm_i, sem0, sem1):
    cid = lax.axis_index("c")
    sid = lax.axis_index("s")
    wid = cid * NUM_SUBCORES + sid
    tile_row = wid // GROUPS
    g = wid % GROUPS
    row0 = tile_row * SUBROWS
    col_g = g * (TCOLS_PER_G * 128)
    bufs = (buf0, buf1)
    sems = (sem0, sem1)
    iota = lax.iota(jnp.int32, 16)

    neg_inf = jnp.full((16,), -jnp.inf, jnp.float32)
    for s in range(SUBROWS):
        acc_m[s, pl.ds(0, 16)] = neg_inf
        acc_i[s, pl.ds(0, 16)] = jnp.zeros((16,), jnp.int32)

    def start_dma(c):
        b = c % 2
        return pltpu.make_async_copy(
            logits_hbm.at[pl.ds(row0, SUBROWS),
                          pl.ds(col_g + CHUNK_STARTS[c] * 128, CHUNK_W)],
            bufs[b], sems[b])

    copies = [None, None]
    copies[0] = start_dma(0)
    copies[0].start()
    for c in range(len(CHUNK_STARTS)):
        if c + 1 < len(CHUNK_STARTS):
            copies[(c + 1) % 2] = start_dma(c + 1)
            copies[(c + 1) % 2].start()
        copies[c % 2].wait()
        buf = bufs[c % 2]
        col0 = col_g + CHUNK_STARTS[c] * 128

        def s_body(s, _, buf=buf, col0=col0):
            def body(j, carry):
                mv, mi, bi = carry
                v = buf[s, pl.ds(j * 16, 16)]
                p = v > mv
                return (jnp.where(p, v, mv), jnp.where(p, bi, mi), bi + 16)
            mv, mi, _ = lax.fori_loop(
                0, VECS, body,
                (acc_m[s, pl.ds(0, 16)], acc_i[s, pl.ds(0, 16)],
                 col0 + iota),
                unroll=8)
            acc_m[s, pl.ds(0, 16)] = mv
            acc_i[s, pl.ds(0, 16)] = mi
            return 0

        lax.fori_loop(0, SUBROWS, s_body, 0)

    @pl.when(g == GROUPS - 1)
    def _():
        pltpu.sync_copy(
            logits_hbm.at[pl.ds(row0, SUBROWS), pl.ds(TAIL_COL, TAIL_W)],
            pbuf)

        def ps_body(s, _):
            def body(j, carry):
                mv, mi, bi = carry
                v = pbuf[s, pl.ds(j * 16, 16)]
                p = v > mv
                return (jnp.where(p, v, mv), jnp.where(p, bi, mi), bi + 16)
            mv, mi, _ = lax.fori_loop(
                0, TAIL_W // 16, body,
                (acc_m[s, pl.ds(0, 16)], acc_i[s, pl.ds(0, 16)],
                 TAIL_COL + iota))
            acc_m[s, pl.ds(0, 16)] = mv
            acc_i[s, pl.ds(0, 16)] = mi
            return 0

        lax.fori_loop(0, SUBROWS, ps_body, 0)

    for s in range(SUBROWS):
        mv, mi = acc_m[s, pl.ds(0, 16)], acc_i[s, pl.ds(0, 16)]
        for sh in (8, 4, 2, 1):
            perm = (iota + sh) & 15
            mv, mi = _combine(mv, mi, _lane_permute(mv, perm),
                              _lane_permute(mi, perm))
        acc_m[s, pl.ds(0, 16)] = mv
        acc_i[s, pl.ds(0, 16)] = mi

    pltpu.sync_copy(acc_m, spm_v.at[sid])
    pltpu.sync_copy(acc_i, spm_i.at[sid])
    plsc.subcore_barrier()

    for t in range(2):
        lrow = sid * 2 + t
        rloc = lax.div(lrow, SUBROWS)
        s = lax.rem(lrow, SUBROWS)
        mv, mi = None, None
        for q in range(GROUPS):
            pltpu.sync_copy(spm_v.at[rloc * GROUPS + q], tmp_v)
            pltpu.sync_copy(spm_i.at[rloc * GROUPS + q], tmp_i)
            v = tmp_v[s, pl.ds(0, 16)]
            i = tmp_i[s, pl.ds(0, 16)]
            if mv is None:
                mv, mi = v, i
            else:
                mv, mi = _combine(mv, mi, v, i)
        res_i[...] = mi
        res_v[...] = mv
        off = (cid * 32 + lrow) * 16
        pltpu.sync_copy(res_i, out_i_hbm.at[pl.ds(off, 16)])
        pltpu.sync_copy(res_v, out_v_hbm.at[pl.ds(off, 16)])


def _tc_body(in_ref, val_ref, idx_ref):
    pid = pl.program_id(0)

    @pl.when(pid == 0)
    def _():
        val_ref[...] = jnp.full((ROWS,), -jnp.inf, jnp.float32)
        idx_ref[...] = jnp.zeros((ROWS,), jnp.int32)

    x = in_ref[...]
    bm = jnp.max(x, axis=1)
    bi = (pid + OFF_BLK) * BLK + jnp.argmax(x, axis=1).astype(jnp.int32)
    cur_v = val_ref[...]
    cur_i = idx_ref[...]
    p = bm > cur_v
    val_ref[...] = jnp.where(p, bm, cur_v)
    idx_ref[...] = jnp.where(p, bi, cur_i)


@jax.jit
def _argmax_split(logits):
    mesh = plsc.VectorSubcoreMesh(core_axis_name="c", subcore_axis_name="s")
    sc_run = pl.kernel(
        _sc_body,
        mesh=mesh,
        out_type=(jax.ShapeDtypeStruct((ROWS * 16,), jnp.int32),
                  jax.ShapeDtypeStruct((ROWS * 16,), jnp.float32)),
        scratch_types=[
            pltpu.VMEM((SUBROWS, CHUNK_W), jnp.float32),
            pltpu.VMEM((SUBROWS, CHUNK_W), jnp.float32),
            pltpu.VMEM((SUBROWS, TAIL_W), jnp.float32),
            pltpu.VMEM((SUBROWS, 128), jnp.float32),
            pltpu.VMEM((SUBROWS, 128), jnp.int32),
            pltpu.VMEM((16,), jnp.int32),
            pltpu.VMEM((16,), jnp.float32),
            pltpu.VMEM((SUBROWS, 128), jnp.float32),
            pltpu.VMEM((SUBROWS, 128), jnp.int32),
            pltpu.VMEM_SHARED((NUM_SUBCORES, SUBROWS, 128), jnp.float32),
            pltpu.VMEM_SHARED((NUM_SUBCORES, SUBROWS, 128), jnp.int32),
            pltpu.SemaphoreType.DMA,
            pltpu.SemaphoreType.DMA,
        ],
    )
    sc_i, sc_v = sc_run(logits)
    sc_i = sc_i[::16]
    sc_v = sc_v[::16]

    tc_v, tc_i = pl.pallas_call(
        _tc_body,
        grid=(TC_STEPS,),
        in_specs=[pl.BlockSpec((ROWS, BLK), lambda i: (0, i + OFF_BLK))],
        out_specs=(pl.BlockSpec((ROWS,), lambda i: (0,)),
                   pl.BlockSpec((ROWS,), lambda i: (0,))),
        out_shape=(jax.ShapeDtypeStruct((ROWS,), jnp.float32),
                   jax.ShapeDtypeStruct((ROWS,), jnp.int32)),
    )(logits)

    p = (tc_v > sc_v) | ((tc_v == sc_v) & (tc_i < sc_i))
    return jnp.where(p, tc_i, sc_i)


def kernel(logits):
    return _argmax_split(logits)
